# baseline (speedup 1.0000x reference)
import jax
import jax.numpy as jnp
from jax.experimental import pallas as pl
from jax.experimental.pallas import tpu as pltpu

N = 10000
BM = 400


def _layer1_body(x_ref, w1_ref, b1_ref, adj_ref, h_ref, xw1_scr):
    @pl.when(pl.program_id(0) == 0)
    def _():
        xw1_scr[...] = jnp.dot(
            x_ref[...], w1_ref[...], preferred_element_type=jnp.float32
        )

    acc = jnp.dot(adj_ref[...], xw1_scr[...], preferred_element_type=jnp.float32)
    h_ref[...] = jnp.maximum(acc + b1_ref[...], 0.0)


def _layer2_body(h_ref, w2_ref, adj_ref, out_ref, hw2_scr):
    @pl.when(pl.program_id(0) == 0)
    def _():
        hw2_scr[...] = jnp.dot(
            h_ref[...], w2_ref[...], preferred_element_type=jnp.float32
        )

    logits = jnp.dot(adj_ref[...], hw2_scr[...], preferred_element_type=jnp.float32)
    m = jnp.max(logits, axis=1, keepdims=True)
    e = jnp.exp(logits - m)
    out_ref[...] = e / jnp.sum(e, axis=1, keepdims=True)


def kernel(x, adj, W1, b1, W2):
    n, nfeat = x.shape
    nhid = W1.shape[1]
    nclass = W2.shape[1]
    b1r = b1.reshape(1, nhid)
    steps = n // BM

    h = pl.pallas_call(
        _layer1_body,
        grid=(steps,),
        in_specs=[
            pl.BlockSpec((n, nfeat), lambda i: (0, 0)),
            pl.BlockSpec((nfeat, nhid), lambda i: (0, 0)),
            pl.BlockSpec((1, nhid), lambda i: (0, 0)),
            pl.BlockSpec((BM, n), lambda i: (i, 0)),
        ],
        out_specs=pl.BlockSpec((BM, nhid), lambda i: (i, 0)),
        out_shape=jax.ShapeDtypeStruct((n, nhid), jnp.float32),
        scratch_shapes=[pltpu.VMEM((n, nhid), jnp.float32)],
    )(x, W1, b1r, adj)

    out = pl.pallas_call(
        _layer2_body,
        grid=(steps,),
        in_specs=[
            pl.BlockSpec((n, nhid), lambda i: (0, 0)),
            pl.BlockSpec((nhid, nclass), lambda i: (0, 0)),
            pl.BlockSpec((BM, n), lambda i: (i, 0)),
        ],
        out_specs=pl.BlockSpec((BM, nclass), lambda i: (i, 0)),
        out_shape=jax.ShapeDtypeStruct((n, nclass), jnp.float32),
        scratch_shapes=[pltpu.VMEM((n, nclass), jnp.float32)],
    )(h, W2, adj)

    return out


# speedup vs baseline: 1.0200x; 1.0200x over previous
import jax
import jax.numpy as jnp
from jax.experimental import pallas as pl
from jax.experimental.pallas import tpu as pltpu

BM = 400


def kernel(x, adj, W1, b1, W2):
    n, nfeat = x.shape
    nhid = W1.shape[1]
    nclass = W2.shape[1]
    b1r = b1.reshape(1, nhid)
    steps = n // BM

    def body(x_ref, w1_ref, b1_ref, w2_ref, adj_ref, out_ref, xw1_scr, hw2_scr):
        s = pl.program_id(0)

        @pl.when(s == 0)
        def _():
            xw1_scr[...] = jnp.dot(
                x_ref[...], w1_ref[...], preferred_element_type=jnp.float32
            )

        @pl.when(s < steps)
        def _():
            acc = jnp.dot(
                adj_ref[...], xw1_scr[...], preferred_element_type=jnp.float32
            )
            hblk = jnp.maximum(acc + b1_ref[...], 0.0)
            hw2_scr[pl.ds(s * BM, BM), :] = jnp.dot(
                hblk, w2_ref[...], preferred_element_type=jnp.float32
            )

        @pl.when(s >= steps)
        def _():
            logits = jnp.dot(
                adj_ref[...], hw2_scr[...], preferred_element_type=jnp.float32
            )
            m = jnp.max(logits, axis=1, keepdims=True)
            e = jnp.exp(logits - m)
            out_ref[...] = e / jnp.sum(e, axis=1, keepdims=True)

    adj_idx = lambda s: (jnp.where(s < steps, s, 2 * steps - 1 - s), 0)
    out_idx = lambda s: (jnp.where(s < steps, steps - 1, 2 * steps - 1 - s), 0)

    return pl.pallas_call(
        body,
        grid=(2 * steps,),
        in_specs=[
            pl.BlockSpec((n, nfeat), lambda s: (0, 0)),
            pl.BlockSpec((nfeat, nhid), lambda s: (0, 0)),
            pl.BlockSpec((1, nhid), lambda s: (0, 0)),
            pl.BlockSpec((nhid, nclass), lambda s: (0, 0)),
            pl.BlockSpec((BM, n), adj_idx),
        ],
        out_specs=pl.BlockSpec((BM, nclass), out_idx),
        out_shape=jax.ShapeDtypeStruct((n, nclass), jnp.float32),
        scratch_shapes=[
            pltpu.VMEM((n, nhid), jnp.float32),
            pltpu.VMEM((n, nclass), jnp.float32),
        ],
        compiler_params=pltpu.CompilerParams(
            vmem_limit_bytes=64 * 1024 * 1024,
        ),
    )(x, W1, b1r, W2, adj)


# speedup vs baseline: 1.0498x; 1.0292x over previous
import jax
import jax.numpy as jnp
from jax.experimental import pallas as pl
from jax.experimental.pallas import tpu as pltpu

BM = 400


def kernel(x, adj, W1, b1, W2):
    n, nfeat = x.shape
    nhid = W1.shape[1]
    nclass = W2.shape[1]
    b1r = b1.reshape(1, nhid)
    steps = n // BM

    def body(x_ref, w1_ref, b1_ref, w2_ref, adj_ref, out_ref,
             xw1_scr, hw2_scr, cache_scr):
        s = pl.program_id(0)

        @pl.when(s == 0)
        def _():
            xw1_scr[...] = jnp.dot(
                x_ref[...], w1_ref[...], preferred_element_type=jnp.float32
            )

        @pl.when(s < steps)
        def _():
            acc = jnp.dot(
                adj_ref[...], xw1_scr[...], preferred_element_type=jnp.float32
            )
            hblk = jnp.maximum(acc + b1_ref[...], 0.0)
            hw2_scr[pl.ds(s * BM, BM), :] = jnp.dot(
                hblk, w2_ref[...], preferred_element_type=jnp.float32
            )

        @pl.when(s == steps - 2)
        def _():
            cache_scr[...] = adj_ref[...]

        def softmax_store(logits):
            m = jnp.max(logits, axis=1, keepdims=True)
            e = jnp.exp(logits - m)
            out_ref[...] = e / jnp.sum(e, axis=1, keepdims=True)

        @pl.when(jnp.logical_and(s >= steps, s != steps + 1))
        def _():
            softmax_store(jnp.dot(
                adj_ref[...], hw2_scr[...], preferred_element_type=jnp.float32
            ))

        @pl.when(s == steps + 1)
        def _():
            softmax_store(jnp.dot(
                cache_scr[...], hw2_scr[...], preferred_element_type=jnp.float32
            ))

    def adj_idx(s):
        b = 2 * steps - 1 - s
        return (jnp.where(s < steps, s, jnp.where(s == steps + 1, steps - 1, b)), 0)

    def out_idx(s):
        return (jnp.where(s < steps, steps - 1, 2 * steps - 1 - s), 0)

    return pl.pallas_call(
        body,
        grid=(2 * steps,),
        in_specs=[
            pl.BlockSpec((n, nfeat), lambda s: (0, 0)),
            pl.BlockSpec((nfeat, nhid), lambda s: (0, 0)),
            pl.BlockSpec((1, nhid), lambda s: (0, 0)),
            pl.BlockSpec((nhid, nclass), lambda s: (0, 0)),
            pl.BlockSpec((BM, n), adj_idx),
        ],
        out_specs=pl.BlockSpec((BM, nclass), out_idx),
        out_shape=jax.ShapeDtypeStruct((n, nclass), jnp.float32),
        scratch_shapes=[
            pltpu.VMEM((n, nhid), jnp.float32),
            pltpu.VMEM((n, nclass), jnp.float32),
            pltpu.VMEM((BM, n), jnp.float32),
        ],
        compiler_params=pltpu.CompilerParams(
            vmem_limit_bytes=64 * 1024 * 1024,
        ),
    )(x, W1, b1r, W2, adj)
